# baseline (speedup 1.0000x reference)
import functools

import jax
import jax.numpy as jnp
from jax.experimental import pallas as pl
from jax.experimental.pallas import tpu as pltpu
from jax.experimental.pallas import tpu_sc as plsc

_GATHER_WINDOW = 128
_BLK = 1024
_CHUNKS = 4


def _sc_gather4(refs, idx2d, chunk, bc):
    d = refs.shape[1]
    mesh = plsc.VectorSubcoreMesh(core_axis_name="core", subcore_axis_name="subcore")
    k = bc // _GATHER_WINDOW
    base = chunk * k

    @functools.partial(
        pl.kernel,
        out_type=jax.ShapeDtypeStruct((4 * bc, d), refs.dtype),
        mesh=mesh,
    )
    def gather_kernel(x_hbm, i_hbm, o_hbm):
        def body(i_vmem, o_vmem):
            pltpu.sync_copy(x_hbm.at[i_vmem.at[0]], o_vmem)

        pltpu.emit_pipeline(
            body,
            grid=(4 * k,),
            in_specs=[pl.BlockSpec((1, _GATHER_WINDOW),
                                   index_map=lambda i: (i // k, i % k + base))],
            out_specs=[pl.BlockSpec((_GATHER_WINDOW, d),
                                    index_map=lambda i: (i, 0))],
            core_axis_name=("core", "subcore"),
            dimension_semantics=(pltpu.PARALLEL,),
        )(i_hbm, o_hbm)

    return gather_kernel(refs, idx2d)


def _loss_kernel(gi, go, gn, gm, win, wout, *rest, inv_scale, has_acc):
    if has_acc:
        acc, out_ref = rest
    else:
        (out_ref,) = rest
    i = pl.program_id(0)
    nsteps = pl.num_programs(0)

    f32 = jnp.float32
    iv = jnp.dot(gi[...], win[...], preferred_element_type=f32, precision=jax.lax.Precision.DEFAULT)
    ov = jnp.dot(go[...], wout[...], preferred_element_type=f32, precision=jax.lax.Precision.DEFAULT)
    nv = jnp.dot(gn[...], win[...], preferred_element_type=f32, precision=jax.lax.Precision.DEFAULT)
    mv = jnp.dot(gm[...], wout[...], preferred_element_type=f32, precision=jax.lax.Precision.DEFAULT)

    eps = 1e-6
    margin = 1.0 - 0.05
    inv_pi = 1.0 / jnp.pi

    n2_i = jnp.sum(iv * iv, axis=1)
    n2_o = jnp.sum(ov * ov, axis=1)
    n2_n = jnp.sum(nv * nv, axis=1)
    n2_m = jnp.sum(mv * mv, axis=1)

    dp_pos = jnp.sum(iv * ov, axis=1)
    dp_neg = jnp.sum(nv * mv, axis=1)

    eps2 = eps * eps
    cos_pos = dp_pos * jax.lax.rsqrt(jnp.maximum(n2_i, eps2) * jnp.maximum(n2_o, eps2))
    cos_neg = dp_neg * jax.lax.rsqrt(jnp.maximum(n2_n, eps2) * jnp.maximum(n2_m, eps2))
    rad_pos = jnp.sqrt(n2_i * n2_o)
    rad_neg = jnp.sqrt(n2_n * n2_m)

    def _arccos(x):
        y = jnp.abs(x)
        p = jnp.float32(-0.0012624911)
        p = p * y + jnp.float32(0.0066700901)
        p = p * y + jnp.float32(-0.0170881256)
        p = p * y + jnp.float32(0.0308918810)
        p = p * y + jnp.float32(-0.0501743046)
        p = p * y + jnp.float32(0.0889789874)
        p = p * y + jnp.float32(-0.2145988016)
        p = p * y + jnp.float32(1.5707963050)
        r = jnp.sqrt(1.0 - y) * p
        return jnp.where(x >= 0, r, jnp.float32(jnp.pi) - r)

    ang_pos = _arccos(margin * cos_pos)
    ang_neg = _arccos(margin * cos_neg)

    z_pos = ang_pos * rad_pos * inv_pi
    z_neg = ang_neg * rad_neg * inv_pi

    u = jnp.exp(-z_pos)
    v = jnp.exp(-z_neg)
    blk_sum = jnp.sum(z_pos + jnp.log((1.0 + u) * (1.0 + v)))

    @pl.when(i == 0)
    def _():
        out_ref[0, 0] = acc[0, 0] if has_acc else 0.0

    out_ref[0, 0] += blk_sum

    if inv_scale is not None:
        @pl.when(i == nsteps - 1)
        def _():
            out_ref[0, 0] = out_ref[0, 0] * inv_scale


def _loss_tc(g, w_in, w_out, batch, acc, inv_scale):
    k = batch // _BLK
    d = g.shape[1]
    row_spec = lambda off: pl.BlockSpec((_BLK, d), lambda i, off=off: (i + off, 0))
    w_spec = pl.BlockSpec((d, w_in.shape[1]), lambda i: (0, 0))
    has_acc = acc is not None
    in_specs = [row_spec(0), row_spec(k), row_spec(2 * k), row_spec(3 * k),
                w_spec, w_spec]
    args = [g, g, g, g, w_in, w_out]
    if has_acc:
        in_specs.append(pl.BlockSpec(memory_space=pltpu.MemorySpace.SMEM))
        args.append(acc)
    return pl.pallas_call(
        functools.partial(_loss_kernel, inv_scale=inv_scale, has_acc=has_acc),
        grid=(k,),
        in_specs=in_specs,
        out_specs=pl.BlockSpec(memory_space=pltpu.MemorySpace.SMEM),
        out_shape=jax.ShapeDtypeStruct((1, 1), jnp.float32),
        compiler_params=pltpu.CompilerParams(
            dimension_semantics=("arbitrary",),
        ),
    )(*args)


def kernel(iword, oword, inword, onword, refs, W_in, W_out):
    batch = iword.shape[0]
    bc = batch // _CHUNKS
    idx2d = jnp.stack([iword, oword, inword, onword]).astype(jnp.int32)
    acc = None
    for c in range(_CHUNKS):
        g = _sc_gather4(refs, idx2d, c, bc)
        acc = _loss_tc(g, W_in, W_out, bc, acc,
                       inv_scale=(1.0 / batch) if c == _CHUNKS - 1 else None)
    return acc.reshape(())


# speedup vs baseline: 1.0531x; 1.0531x over previous
import functools

import jax
import jax.numpy as jnp
from jax.experimental import pallas as pl
from jax.experimental.pallas import tpu as pltpu
from jax.experimental.pallas import tpu_sc as plsc

_GATHER_WINDOW = 128
_BLK = 1024
_CHUNKS = 2


def _sc_gather4(refs, idx2d, chunk, bc):
    d = refs.shape[1]
    mesh = plsc.VectorSubcoreMesh(core_axis_name="core", subcore_axis_name="subcore")
    k = bc // _GATHER_WINDOW
    base = chunk * k

    @functools.partial(
        pl.kernel,
        out_type=jax.ShapeDtypeStruct((4 * bc, d), refs.dtype),
        mesh=mesh,
    )
    def gather_kernel(x_hbm, i_hbm, o_hbm):
        def body(i_vmem, o_vmem):
            pltpu.sync_copy(x_hbm.at[i_vmem.at[0]], o_vmem)

        pltpu.emit_pipeline(
            body,
            grid=(4 * k,),
            in_specs=[pl.BlockSpec((1, _GATHER_WINDOW),
                                   index_map=lambda i: (i // k, i % k + base))],
            out_specs=[pl.BlockSpec((_GATHER_WINDOW, d),
                                    index_map=lambda i: (i, 0))],
            core_axis_name=("core", "subcore"),
            dimension_semantics=(pltpu.PARALLEL,),
        )(i_hbm, o_hbm)

    return gather_kernel(refs, idx2d)


def _loss_kernel(gi, go, gn, gm, win, wout, *rest, inv_scale, has_acc):
    if has_acc:
        acc, out_ref = rest
    else:
        (out_ref,) = rest
    i = pl.program_id(0)
    nsteps = pl.num_programs(0)

    f32 = jnp.float32
    iv = jnp.dot(gi[...], win[...], preferred_element_type=f32, precision=jax.lax.Precision.DEFAULT)
    ov = jnp.dot(go[...], wout[...], preferred_element_type=f32, precision=jax.lax.Precision.DEFAULT)
    nv = jnp.dot(gn[...], win[...], preferred_element_type=f32, precision=jax.lax.Precision.DEFAULT)
    mv = jnp.dot(gm[...], wout[...], preferred_element_type=f32, precision=jax.lax.Precision.DEFAULT)

    eps = 1e-6
    margin = 1.0 - 0.05
    inv_pi = 1.0 / jnp.pi

    n2_i = jnp.sum(iv * iv, axis=1)
    n2_o = jnp.sum(ov * ov, axis=1)
    n2_n = jnp.sum(nv * nv, axis=1)
    n2_m = jnp.sum(mv * mv, axis=1)

    dp_pos = jnp.sum(iv * ov, axis=1)
    dp_neg = jnp.sum(nv * mv, axis=1)

    eps2 = eps * eps
    cos_pos = dp_pos * jax.lax.rsqrt(jnp.maximum(n2_i, eps2) * jnp.maximum(n2_o, eps2))
    cos_neg = dp_neg * jax.lax.rsqrt(jnp.maximum(n2_n, eps2) * jnp.maximum(n2_m, eps2))
    rad_pos = jnp.sqrt(n2_i * n2_o)
    rad_neg = jnp.sqrt(n2_n * n2_m)

    def _arccos(x):
        y = jnp.abs(x)
        p = jnp.float32(-0.0012624911)
        p = p * y + jnp.float32(0.0066700901)
        p = p * y + jnp.float32(-0.0170881256)
        p = p * y + jnp.float32(0.0308918810)
        p = p * y + jnp.float32(-0.0501743046)
        p = p * y + jnp.float32(0.0889789874)
        p = p * y + jnp.float32(-0.2145988016)
        p = p * y + jnp.float32(1.5707963050)
        r = jnp.sqrt(1.0 - y) * p
        return jnp.where(x >= 0, r, jnp.float32(jnp.pi) - r)

    ang_pos = _arccos(margin * cos_pos)
    ang_neg = _arccos(margin * cos_neg)

    z_pos = ang_pos * rad_pos * inv_pi
    z_neg = ang_neg * rad_neg * inv_pi

    u = jnp.exp(-z_pos)
    v = jnp.exp(-z_neg)
    blk_sum = jnp.sum(z_pos + jnp.log((1.0 + u) * (1.0 + v)))

    @pl.when(i == 0)
    def _():
        out_ref[0, 0] = acc[0, 0] if has_acc else 0.0

    out_ref[0, 0] += blk_sum

    if inv_scale is not None:
        @pl.when(i == nsteps - 1)
        def _():
            out_ref[0, 0] = out_ref[0, 0] * inv_scale


def _loss_tc(g, w_in, w_out, batch, acc, inv_scale):
    k = batch // _BLK
    d = g.shape[1]
    row_spec = lambda off: pl.BlockSpec((_BLK, d), lambda i, off=off: (i + off, 0))
    w_spec = pl.BlockSpec((d, w_in.shape[1]), lambda i: (0, 0))
    has_acc = acc is not None
    in_specs = [row_spec(0), row_spec(k), row_spec(2 * k), row_spec(3 * k),
                w_spec, w_spec]
    args = [g, g, g, g, w_in, w_out]
    if has_acc:
        in_specs.append(pl.BlockSpec(memory_space=pltpu.MemorySpace.SMEM))
        args.append(acc)
    return pl.pallas_call(
        functools.partial(_loss_kernel, inv_scale=inv_scale, has_acc=has_acc),
        grid=(k,),
        in_specs=in_specs,
        out_specs=pl.BlockSpec(memory_space=pltpu.MemorySpace.SMEM),
        out_shape=jax.ShapeDtypeStruct((1, 1), jnp.float32),
        compiler_params=pltpu.CompilerParams(
            dimension_semantics=("arbitrary",),
        ),
    )(*args)


def kernel(iword, oword, inword, onword, refs, W_in, W_out):
    batch = iword.shape[0]
    bc = batch // _CHUNKS
    idx2d = jnp.stack([iword, oword, inword, onword]).astype(jnp.int32)
    acc = None
    for c in range(_CHUNKS):
        g = _sc_gather4(refs, idx2d, c, bc)
        acc = _loss_tc(g, W_in, W_out, bc, acc,
                       inv_scale=(1.0 / batch) if c == _CHUNKS - 1 else None)
    return acc.reshape(())
